# baseline (speedup 1.0000x reference)
import functools

import jax
import jax.numpy as jnp
from jax import lax
from jax.experimental import pallas as pl
from jax.experimental.pallas import tpu as pltpu
from jax.experimental.pallas import tpu_sc as plsc

VOCAB = 1_000_000
EMBED_DIM = 64
BATCH = 4096
HIST = 200

_NUM_WORKERS = 32
_B_PER_W = BATCH // _NUM_WORKERS
_SPLIT = 128
_REST = HIST - _SPLIT
_NBUF = 8
_DG = 5


def _make_sc_gather():
    mesh = plsc.VectorSubcoreMesh(core_axis_name="c", subcore_axis_name="s")

    @functools.partial(
        pl.kernel,
        mesh=mesh,
        out_type=jax.ShapeDtypeStruct((BATCH, HIST, EMBED_DIM), jnp.float32),
        scratch_types=[
            pltpu.VMEM((_B_PER_W, HIST), jnp.int32),
            pltpu.VMEM((_NBUF, HIST, EMBED_DIM), jnp.float32),
            pltpu.SemaphoreType.DMA,
            pltpu.SemaphoreType.DMA,
        ],
        compiler_params=pltpu.CompilerParams(use_tc_tiling_on_sc=False),
    )
    def gather_kernel(idx_hbm, table_hbm, out_hbm, idx_v, rows_v, gsem, osem):
        nc = lax.axis_size("c")
        wid = lax.axis_index("s") * nc + lax.axis_index("c")
        w_b0 = pl.multiple_of(wid * _B_PER_W, _B_PER_W)

        pltpu.sync_copy(idx_hbm.at[pl.ds(w_b0, _B_PER_W), :], idx_v)

        def buf_of(c):
            return lax.rem(c, _NBUF) if not isinstance(c, int) else c % _NBUF

        def gather_copies(c):
            buf = buf_of(c)
            return (
                pltpu.make_async_copy(
                    table_hbm.at[idx_v.at[c, pl.ds(0, _SPLIT)]],
                    rows_v.at[buf, pl.ds(0, _SPLIT), :],
                    gsem,
                ),
                pltpu.make_async_copy(
                    table_hbm.at[idx_v.at[c, pl.ds(_SPLIT, _REST)]],
                    rows_v.at[buf, pl.ds(_SPLIT, _REST), :],
                    gsem,
                ),
            )

        def out_copy(c):
            return pltpu.make_async_copy(
                rows_v.at[buf_of(c)], out_hbm.at[w_b0 + c], osem
            )

        def fire_gathers(c):
            for cp in gather_copies(c):
                cp.start()

        def wait_gathers(c):
            for cp in gather_copies(c):
                cp.wait()

        for t in range(_DG):
            fire_gathers(t)
        for t in range(_DG, _NBUF):
            wait_gathers(t - _DG)
            out_copy(t - _DG).start()
            fire_gathers(t)

        def body(t, carry):
            wait_gathers(t - _DG)
            out_copy(t - _DG).start()
            out_copy(t - _NBUF).wait()
            fire_gathers(t)
            return carry

        lax.fori_loop(_NBUF, _B_PER_W, body, 0)

        for t in range(_B_PER_W, _B_PER_W + _DG):
            wait_gathers(t - _DG)
            out_copy(t - _DG).start()
        for c in range(_B_PER_W - _NBUF, _B_PER_W):
            out_copy(c).wait()

    return gather_kernel


@functools.cache
def _sc_gather():
    return _make_sc_gather()


def kernel(words, weight):
    return _sc_gather()(words.astype(jnp.int32), weight)
